# baseline (speedup 1.0000x reference)
import functools

import jax
import jax.numpy as jnp
import numpy as np
from jax import lax
from jax.experimental import pallas as pl
from jax.experimental.pallas import tpu as pltpu
from jax.experimental.pallas import tpu_sc as plsc

_B = 4096
_K = 32
_D = 512
_HD = _D // 2
_OUT_D = 2 * _D
_LANES = 16
_NC = 2
_NS = 16
_NW = _NC * _NS
_BPW = _B // _NW
_BS = 4
_STEPS = _BPW // _BS
_NCH = _HD // _LANES
_HCH = _NCH // 2

_mesh = plsc.VectorSubcoreMesh(core_axis_name="c", subcore_axis_name="s")


@functools.partial(
    pl.kernel,
    out_type=jax.ShapeDtypeStruct((_B, _OUT_D), jnp.float32),
    mesh=_mesh,
    scratch_types=[
        pltpu.VMEM((_BPW * _K,), jnp.int32),
        pltpu.VMEM((3, _BS * _K, _HD), jnp.int32),
        pltpu.VMEM((3, _BS, _D), jnp.float32),
        pltpu.VMEM((3, _BS, _OUT_D), jnp.float32),
        pltpu.SemaphoreType.DMA,
        pltpu.SemaphoreType.DMA,
        pltpu.SemaphoreType.DMA,
    ],
)
def _intra_agg(emb_hbm, idx_hbm, self_hbm, out_hbm,
               idx_v, rows_v, self_v, out_v, gsem, ssem, osem):
    wid = lax.axis_index("s") * _NC + lax.axis_index("c")
    base = wid * _BPW

    pltpu.sync_copy(idx_hbm.at[pl.ds(base * _K, _BPW * _K)], idx_v)

    def issue(step, buf):
        pltpu.async_copy(
            emb_hbm.at[idx_v.at[pl.ds(step * (_BS * _K), _BS * _K)]],
            rows_v.at[buf], gsem)
        pltpu.async_copy(
            self_hbm.at[pl.ds(base + step * _BS, _BS)], self_v.at[buf], ssem)

    def wait_in(buf):
        pltpu.make_async_copy(
            emb_hbm.at[pl.ds(0, _BS * _K)], rows_v.at[buf], gsem).wait()
        pltpu.make_async_copy(
            self_hbm.at[pl.ds(0, _BS)], self_v.at[buf], ssem).wait()

    def drain_out(buf):
        pltpu.make_async_copy(
            out_v.at[buf], out_hbm.at[pl.ds(base, _BS)], osem).wait()

    rows_bf = rows_v.bitcast(jnp.bfloat16)

    def compute(step, buf):
        for j in range(_BS):
            r0 = j * _K
            for h in range(2):
                c0 = h * _HCH

                def kbody(kk, accs, _r0=r0, _c0=c0, _buf=buf):
                    lo, hi = accs
                    nlo, nhi = [], []
                    for c in range(_HCH):
                        sl = pl.ds((_c0 + c) * _LANES, _LANES)
                        va = rows_bf[_buf, pl.ds(4 * kk + 2 * _r0, 2), sl]
                        vb = rows_bf[_buf, pl.ds(4 * kk + 2 * _r0 + 2, 2),
                                     sl]
                        m = (va + vb).astype(jnp.float32)
                        nlo.append(lo[c] + m[0])
                        nhi.append(hi[c] + m[1])
                    return tuple(nlo), tuple(nhi)

                zeros = tuple(jnp.zeros((_LANES,), jnp.float32)
                              for _ in range(_HCH))
                lo, hi = lax.fori_loop(0, _K // 2, kbody, (zeros, zeros))
                for c in range(_HCH):
                    off = (c0 + c) * _LANES
                    ml = lo[c] * (1.0 / _K)
                    mh = hi[c] * (1.0 / _K)
                    out_v[buf, j, pl.ds(off, _LANES)] = ml
                    out_v[buf, j, pl.ds(_HD + off, _LANES)] = mh
                    out_v[buf, j, pl.ds(_D + off, _LANES)] = (
                        self_v[buf, j, pl.ds(off, _LANES)] - ml)
                    out_v[buf, j, pl.ds(_D + _HD + off, _LANES)] = (
                        self_v[buf, j, pl.ds(_HD + off, _LANES)] - mh)
        pltpu.async_copy(
            out_v.at[buf], out_hbm.at[pl.ds(base + step * _BS, _BS)], osem)

    def body(step, b):
        @pl.when(step + 2 < _STEPS)
        def _():
            issue(step + 2, (b + 2) % 3)

        wait_in(b)

        @pl.when(step >= 3)
        def _():
            drain_out(b)

        compute(step, b)

    issue(0, 0)
    issue(1, 1)

    def outer(i, carry):
        for b in range(3):
            body(i * 3 + b, b)
        return carry

    lax.fori_loop(0, (_STEPS - 2) // 3, outer, 0)
    body(_STEPS - 2, 0)
    body(_STEPS - 1, 1)
    drain_out(2)
    drain_out(0)
    drain_out(1)


def kernel(embedding, neighbor_idx, self_feats):
    au = jax.lax.bitcast_convert_type(embedding[:, :_HD], jnp.uint32)
    bu = jax.lax.bitcast_convert_type(embedding[:, _HD:], jnp.uint32)
    ar = au + jnp.uint32(0x7FFF) + ((au >> 16) & jnp.uint32(1))
    br = bu + jnp.uint32(0x7FFF) + ((bu >> 16) & jnp.uint32(1))
    packed = jax.lax.bitcast_convert_type(
        (ar >> 16) | (br & jnp.uint32(0xFFFF0000)), jnp.int32)
    return _intra_agg(packed, neighbor_idx.reshape(-1), self_feats)


# speedup vs baseline: 1.0075x; 1.0075x over previous
import functools

import jax
import jax.numpy as jnp
import numpy as np
from jax import lax
from jax.experimental import pallas as pl
from jax.experimental.pallas import tpu as pltpu
from jax.experimental.pallas import tpu_sc as plsc

_B = 4096
_K = 32
_D = 512
_HD = _D // 2
_OUT_D = 2 * _D
_LANES = 16
_NC = 2
_NS = 16
_NW = _NC * _NS
_BPW = _B // _NW
_BS = 2
_STEPS = _BPW // _BS
_NCH = _HD // _LANES
_HCH = _NCH // 2

_mesh = plsc.VectorSubcoreMesh(core_axis_name="c", subcore_axis_name="s")


@functools.partial(
    pl.kernel,
    out_type=jax.ShapeDtypeStruct((_B, _OUT_D), jnp.float32),
    mesh=_mesh,
    scratch_types=[
        pltpu.VMEM((_BPW * _K,), jnp.int32),
        pltpu.VMEM((6, _BS * _K, _HD), jnp.int32),
        pltpu.VMEM((6, _BS, _D), jnp.float32),
        pltpu.VMEM((6, _BS, _OUT_D), jnp.float32),
        pltpu.SemaphoreType.DMA,
        pltpu.SemaphoreType.DMA,
        pltpu.SemaphoreType.DMA,
    ],
)
def _intra_agg(emb_hbm, idx_hbm, self_hbm, out_hbm,
               idx_v, rows_v, self_v, out_v, gsem, ssem, osem):
    wid = lax.axis_index("s") * _NC + lax.axis_index("c")
    base = wid * _BPW

    pltpu.sync_copy(idx_hbm.at[pl.ds(base * _K, _BPW * _K)], idx_v)

    def issue(step, buf):
        pltpu.async_copy(
            emb_hbm.at[idx_v.at[pl.ds(step * (_BS * _K), _BS * _K)]],
            rows_v.at[buf], gsem)
        pltpu.async_copy(
            self_hbm.at[pl.ds(base + step * _BS, _BS)], self_v.at[buf], ssem)

    def wait_in(buf):
        pltpu.make_async_copy(
            emb_hbm.at[pl.ds(0, _BS * _K)], rows_v.at[buf], gsem).wait()
        pltpu.make_async_copy(
            self_hbm.at[pl.ds(0, _BS)], self_v.at[buf], ssem).wait()

    def drain_out(buf):
        pltpu.make_async_copy(
            out_v.at[buf], out_hbm.at[pl.ds(base, _BS)], osem).wait()

    rows_bf = rows_v.bitcast(jnp.bfloat16)

    def compute(step, buf):
        for j in range(_BS):
            r0 = j * _K
            for h in range(2):
                c0 = h * _HCH

                def kbody(kk, accs, _r0=r0, _c0=c0, _buf=buf):
                    lo, hi = accs
                    nlo, nhi = [], []
                    for c in range(_HCH):
                        sl = pl.ds((_c0 + c) * _LANES, _LANES)
                        va = rows_bf[_buf, pl.ds(4 * kk + 2 * _r0, 2), sl]
                        vb = rows_bf[_buf, pl.ds(4 * kk + 2 * _r0 + 2, 2),
                                     sl]
                        m = (va + vb).astype(jnp.float32)
                        nlo.append(lo[c] + m[0])
                        nhi.append(hi[c] + m[1])
                    return tuple(nlo), tuple(nhi)

                zeros = tuple(jnp.zeros((_LANES,), jnp.float32)
                              for _ in range(_HCH))
                lo, hi = lax.fori_loop(0, _K // 2, kbody, (zeros, zeros))
                for c in range(_HCH):
                    off = (c0 + c) * _LANES
                    ml = lo[c] * (1.0 / _K)
                    mh = hi[c] * (1.0 / _K)
                    out_v[buf, j, pl.ds(off, _LANES)] = ml
                    out_v[buf, j, pl.ds(_HD + off, _LANES)] = mh
                    out_v[buf, j, pl.ds(_D + off, _LANES)] = (
                        self_v[buf, j, pl.ds(off, _LANES)] - ml)
                    out_v[buf, j, pl.ds(_D + _HD + off, _LANES)] = (
                        self_v[buf, j, pl.ds(_HD + off, _LANES)] - mh)
        pltpu.async_copy(
            out_v.at[buf], out_hbm.at[pl.ds(base + step * _BS, _BS)], osem)

    def body(step, b):
        @pl.when(step + 4 < _STEPS)
        def _():
            issue(step + 4, (b + 4) % 6)

        wait_in(b)

        @pl.when(step >= 6)
        def _():
            drain_out(b)

        compute(step, b)

    for p in range(4):
        issue(p, p)

    def outer(i, carry):
        for b in range(6):
            body(i * 6 + b, b)
        return carry

    lax.fori_loop(0, (_STEPS - 4) // 6, outer, 0)
    for t in range(4):
        body(_STEPS - 4 + t, t)
    for t in range(6):
        drain_out(t)


def kernel(embedding, neighbor_idx, self_feats):
    au = jax.lax.bitcast_convert_type(embedding[:, :_HD], jnp.uint32)
    bu = jax.lax.bitcast_convert_type(embedding[:, _HD:], jnp.uint32)
    ar = au + jnp.uint32(0x7FFF) + ((au >> 16) & jnp.uint32(1))
    br = bu + jnp.uint32(0x7FFF) + ((bu >> 16) & jnp.uint32(1))
    packed = jax.lax.bitcast_convert_type(
        (ar >> 16) | (br & jnp.uint32(0xFFFF0000)), jnp.int32)
    return _intra_agg(packed, neighbor_idx.reshape(-1), self_feats)
